# baseline (speedup 1.0000x reference)
import jax
import jax.numpy as jnp
from jax.experimental import pallas as pl

B, T, D = 4, 2048, 768
E = 8
K = 2
N = B * T
BM = 1024


def _moe_block_kernel(x_ref, gw_ref, gb_ref, ew_ref, eb_ref, out_ref):
    x = x_ref[...]
    logits = jax.lax.dot_general(
        x, gw_ref[...], (((1,), (1,)), ((), ())),
        preferred_element_type=jnp.float32,
    ) + gb_ref[...]
    rank = jnp.zeros((BM, E), dtype=jnp.int32)
    col = jax.lax.broadcasted_iota(jnp.int32, (BM, E), 1)
    for j in range(E):
        lj = logits[:, j:j + 1]
        beats = (lj > logits) | ((lj == logits) & (j < col))
        rank = rank + beats.astype(jnp.int32)
    combine = jnp.where(rank < K, jax.nn.sigmoid(logits), 0.0)
    acc = jax.lax.dot_general(
        combine, eb_ref[...], (((1,), (0,)), ((), ())),
        preferred_element_type=jnp.float32,
    )
    for e in range(E):
        xe = combine[:, e:e + 1] * x
        acc = acc + jax.lax.dot_general(
            xe, ew_ref[e], (((1,), (0,)), ((), ())),
            preferred_element_type=jnp.float32,
        )
    out_ref[...] = acc


def kernel(inputs, gate_W, gate_b, expert_W, expert_b):
    x = inputs.reshape(N, D)
    gb = gate_b.reshape(1, E)
    expert_W = expert_W.transpose(0, 2, 1)
    out = pl.pallas_call(
        _moe_block_kernel,
        grid=(N // BM,),
        in_specs=[
            pl.BlockSpec((BM, D), lambda i: (i, 0)),
            pl.BlockSpec((E, D), lambda i: (0, 0)),
            pl.BlockSpec((1, E), lambda i: (0, 0)),
            pl.BlockSpec((E, D, D), lambda i: (0, 0, 0)),
            pl.BlockSpec((E, D), lambda i: (0, 0)),
        ],
        out_specs=pl.BlockSpec((BM, D), lambda i: (i, 0)),
        out_shape=jax.ShapeDtypeStruct((N, D), jnp.float32),
    )(x, gate_W, gb, expert_W, expert_b)
    return out.reshape(B, T, D)


# speedup vs baseline: 1.3211x; 1.3211x over previous
import jax
import jax.numpy as jnp
from jax.experimental import pallas as pl

B, T, D = 4, 2048, 768
E = 8
K = 2
N = B * T
BM = 1024


def _moe_block_kernel(x_ref, gw_ref, gb_ref, ew_ref, eb_ref, out_ref):
    x = x_ref[...]
    logits = jax.lax.dot_general(
        x, gw_ref[...], (((1,), (1,)), ((), ())),
        preferred_element_type=jnp.float32,
    ) + gb_ref[...]
    rank = jnp.zeros((BM, E), dtype=jnp.int32)
    col = jax.lax.broadcasted_iota(jnp.int32, (BM, E), 1)
    for j in range(E):
        lj = logits[:, j:j + 1]
        beats = (lj > logits) | ((lj == logits) & (j < col))
        rank = rank + beats.astype(jnp.int32)
    combine = jnp.where(rank < K, jax.nn.sigmoid(logits), 0.0)
    acc = jax.lax.dot_general(
        combine, eb_ref[...], (((1,), (0,)), ((), ())),
        preferred_element_type=jnp.float32,
    )
    for e in range(E):
        xe = combine[:, e:e + 1] * x
        acc = acc + jax.lax.dot_general(
            xe, ew_ref[e], (((1,), (1,)), ((), ())),
            preferred_element_type=jnp.float32,
        )
    out_ref[...] = acc


def kernel(inputs, gate_W, gate_b, expert_W, expert_b):
    x = inputs.reshape(N, D)
    gb = gate_b.reshape(1, E)
    out = pl.pallas_call(
        _moe_block_kernel,
        grid=(N // BM,),
        in_specs=[
            pl.BlockSpec((BM, D), lambda i: (i, 0)),
            pl.BlockSpec((E, D), lambda i: (0, 0)),
            pl.BlockSpec((1, E), lambda i: (0, 0)),
            pl.BlockSpec((E, D, D), lambda i: (0, 0, 0)),
            pl.BlockSpec((E, D), lambda i: (0, 0)),
        ],
        out_specs=pl.BlockSpec((BM, D), lambda i: (i, 0)),
        out_shape=jax.ShapeDtypeStruct((N, D), jnp.float32),
    )(x, gate_W, gb, expert_W, expert_b)
    return out.reshape(B, T, D)
